# baseline (speedup 1.0000x reference)
import functools

import jax
import jax.numpy as jnp
from jax import lax
from jax.experimental import pallas as pl
from jax.experimental.pallas import tpu as pltpu
from jax.experimental.pallas import tpu_sc as plsc

NUM_CORES = 2
NUM_SUBCORES = 16
NUM_WORKERS = NUM_CORES * NUM_SUBCORES
CHUNK = 800
NBUF = 4
CHUNK_A = 504
UNROLL_A = 8


def _relayout_kernel(v, d):
    rows_per_w = (v // NUM_WORKERS) // 8 * 8
    n_chunks = rows_per_w // CHUNK_A
    rem = v - rows_per_w * NUM_WORKERS
    mesh = plsc.VectorSubcoreMesh(core_axis_name="c", subcore_axis_name="s")

    @functools.partial(
        pl.kernel,
        out_type=jax.ShapeDtypeStruct((v * d,), jnp.float32),
        mesh=mesh,
        scratch_types=[
            pltpu.VMEM((CHUNK_A, d), jnp.float32),
            pltpu.VMEM((CHUNK_A * d,), jnp.float32),
        ],
    )
    def k(w_hbm, wlin_hbm, pad_v, dense_v):
        wid = lax.axis_index("s") * NUM_CORES + lax.axis_index("c")
        base = wid * rows_per_w

        def do_chunk(r0, nrows):
            pltpu.sync_copy(w_hbm.at[pl.ds(r0, nrows)], pad_v.at[pl.ds(0, nrows)])

            def rows(i, c):
                for u in range(UNROLL_A):
                    r = i * UNROLL_A + u
                    dense_v[pl.ds(r * d, 16)] = pad_v[r, pl.ds(0, 16)]
                    dense_v[pl.ds(r * d + 16, 16)] = pad_v[r, pl.ds(16, 16)]
                return c

            lax.fori_loop(0, nrows // UNROLL_A, rows, 0)
            pltpu.sync_copy(
                dense_v.at[pl.ds(0, nrows * d)], wlin_hbm.at[pl.ds(r0 * d, nrows * d)]
            )

        def body(g, carry):
            do_chunk(base + g * CHUNK_A, CHUNK_A)
            return carry

        lax.fori_loop(0, n_chunks, body, 0)
        if rem:
            @pl.when(wid == 0)
            def _():
                do_chunk(rows_per_w * NUM_WORKERS, rem)

    return k


def _gather_kernel(flat_n, d):
    per_w = flat_n // NUM_WORKERS
    n_chunks = per_w // CHUNK
    mesh = plsc.VectorSubcoreMesh(core_axis_name="c", subcore_axis_name="s")

    @functools.partial(
        pl.kernel,
        out_type=jax.ShapeDtypeStruct((flat_n, d), jnp.float32),
        mesh=mesh,
        scratch_types=[pltpu.VMEM((per_w,), jnp.int32)]
        + [pltpu.VMEM((CHUNK, d), jnp.float32) for _ in range(NBUF)]
        + [pltpu.SemaphoreType.DMA for _ in range(NBUF + 1)],
        compiler_params=pltpu.CompilerParams(use_tc_tiling_on_sc=False),
    )
    def k(idx_hbm, w_hbm, out_hbm, idx_v, *scratch):
        bufs = scratch[:NBUF]
        gsems = scratch[NBUF : 2 * NBUF]
        wsem = scratch[2 * NBUF]
        wid = lax.axis_index("s") * NUM_CORES + lax.axis_index("c")
        base_w = wid * per_w
        pltpu.sync_copy(idx_hbm.at[pl.ds(base_w, per_w)], idx_v)

        def fire_gather(g):
            b = g % NBUF
            return pltpu.async_copy(
                w_hbm.at[idx_v.at[pl.ds(g * CHUNK, CHUNK)]], bufs[b], gsems[b]
            )

        gathers = [fire_gather(g) for g in range(NBUF)]
        for g in range(n_chunks):
            b = g % NBUF
            gathers[b].wait()
            wb = pltpu.async_copy(
                bufs[b], out_hbm.at[pl.ds(base_w + g * CHUNK, CHUNK)], wsem
            )
            wb.wait()
            if g + NBUF < n_chunks:
                gathers[b] = fire_gather(g + NBUF)

    return k


def kernel(idx, W):
    B, L = idx.shape
    V, D = W.shape
    flat = idx.reshape(-1).astype(jnp.int32)
    w_lin = _relayout_kernel(V, D)(W).reshape(V, D)
    out = _gather_kernel(B * L, D)(flat, w_lin)
    return out.reshape(B, L, D)


# speedup vs baseline: 1.1263x; 1.1263x over previous
import functools

import jax
import jax.numpy as jnp
from jax import lax
from jax.experimental import pallas as pl
from jax.experimental.pallas import tpu as pltpu
from jax.experimental.pallas import tpu_sc as plsc

NUM_CORES = 2
NUM_SUBCORES = 16
NUM_WORKERS = NUM_CORES * NUM_SUBCORES
CHUNK = 800
NBUF = 4
CHUNK_A = 504
UNROLL_A = 8


def _relayout_kernel(v, d):
    rows_per_w = (v // NUM_WORKERS) // 8 * 8
    n_chunks = rows_per_w // CHUNK_A
    rem = v - rows_per_w * NUM_WORKERS
    mesh = plsc.VectorSubcoreMesh(core_axis_name="c", subcore_axis_name="s")

    @functools.partial(
        pl.kernel,
        out_type=jax.ShapeDtypeStruct((v * d,), jnp.float32),
        mesh=mesh,
        scratch_types=[
            pltpu.VMEM((CHUNK_A, d), jnp.float32),
            pltpu.VMEM((CHUNK_A * d,), jnp.float32),
        ],
    )
    def k(w_hbm, wlin_hbm, pad_v, dense_v):
        wid = lax.axis_index("s") * NUM_CORES + lax.axis_index("c")
        base = wid * rows_per_w

        def do_chunk(r0, nrows):
            pltpu.sync_copy(w_hbm.at[pl.ds(r0, nrows)], pad_v.at[pl.ds(0, nrows)])

            @functools.partial(plsc.parallel_loop, 0, nrows, unroll=UNROLL_A)
            def rows(r):
                dense_v[pl.ds(r * d, 16)] = pad_v[r, pl.ds(0, 16)]
                dense_v[pl.ds(r * d + 16, 16)] = pad_v[r, pl.ds(16, 16)]
            pltpu.sync_copy(
                dense_v.at[pl.ds(0, nrows * d)], wlin_hbm.at[pl.ds(r0 * d, nrows * d)]
            )

        def body(g, carry):
            do_chunk(base + g * CHUNK_A, CHUNK_A)
            return carry

        lax.fori_loop(0, n_chunks, body, 0)
        if rem:
            @pl.when(wid == 0)
            def _():
                do_chunk(rows_per_w * NUM_WORKERS, rem)

    return k


def _gather_kernel(flat_n, d):
    per_w = flat_n // NUM_WORKERS
    n_chunks = per_w // CHUNK
    mesh = plsc.VectorSubcoreMesh(core_axis_name="c", subcore_axis_name="s")

    @functools.partial(
        pl.kernel,
        out_type=jax.ShapeDtypeStruct((flat_n, d), jnp.float32),
        mesh=mesh,
        scratch_types=[pltpu.VMEM((per_w,), jnp.int32)]
        + [pltpu.VMEM((CHUNK, d), jnp.float32) for _ in range(NBUF)]
        + [pltpu.SemaphoreType.DMA for _ in range(NBUF + 1)],
        compiler_params=pltpu.CompilerParams(use_tc_tiling_on_sc=False),
    )
    def k(idx_hbm, w_hbm, out_hbm, idx_v, *scratch):
        bufs = scratch[:NBUF]
        gsems = scratch[NBUF : 2 * NBUF]
        wsem = scratch[2 * NBUF]
        wid = lax.axis_index("s") * NUM_CORES + lax.axis_index("c")
        base_w = wid * per_w
        pltpu.sync_copy(idx_hbm.at[pl.ds(base_w, per_w)], idx_v)

        def fire_gather(g):
            b = g % NBUF
            return pltpu.async_copy(
                w_hbm.at[idx_v.at[pl.ds(g * CHUNK, CHUNK)]], bufs[b], gsems[b]
            )

        gathers = [fire_gather(g) for g in range(NBUF)]
        for g in range(n_chunks):
            b = g % NBUF
            gathers[b].wait()
            wb = pltpu.async_copy(
                bufs[b], out_hbm.at[pl.ds(base_w + g * CHUNK, CHUNK)], wsem
            )
            wb.wait()
            if g + NBUF < n_chunks:
                gathers[b] = fire_gather(g + NBUF)

    return k


def kernel(idx, W):
    B, L = idx.shape
    V, D = W.shape
    flat = idx.reshape(-1).astype(jnp.int32)
    w_lin = _relayout_kernel(V, D)(W).reshape(V, D)
    out = _gather_kernel(B * L, D)(flat, w_lin)
    return out.reshape(B, L, D)


# speedup vs baseline: 1.1563x; 1.0266x over previous
import functools

import jax
import jax.numpy as jnp
from jax import lax
from jax.experimental import pallas as pl
from jax.experimental.pallas import tpu as pltpu
from jax.experimental.pallas import tpu_sc as plsc

NUM_CORES = 2
NUM_SUBCORES = 16
NUM_WORKERS = NUM_CORES * NUM_SUBCORES
CHUNK = 800
NBUF = 4


def _gather_kernel(flat_n, d):
    per_w = flat_n // NUM_WORKERS
    n_chunks = per_w // CHUNK
    mesh = plsc.VectorSubcoreMesh(core_axis_name="c", subcore_axis_name="s")

    @functools.partial(
        pl.kernel,
        out_type=jax.ShapeDtypeStruct((flat_n, d), jnp.float32),
        mesh=mesh,
        scratch_types=[pltpu.VMEM((per_w,), jnp.int32)]
        + [pltpu.VMEM((CHUNK, d), jnp.float32) for _ in range(NBUF)]
        + [pltpu.SemaphoreType.DMA for _ in range(NBUF + 1)],
        compiler_params=pltpu.CompilerParams(use_tc_tiling_on_sc=False),
    )
    def k(idx_hbm, w_hbm, out_hbm, idx_v, *scratch):
        bufs = scratch[:NBUF]
        gsems = scratch[NBUF : 2 * NBUF]
        wsem = scratch[2 * NBUF]
        wid = lax.axis_index("s") * NUM_CORES + lax.axis_index("c")
        base_w = wid * per_w
        pltpu.sync_copy(idx_hbm.at[pl.ds(base_w, per_w)], idx_v)

        def fire_gather(g):
            b = g % NBUF
            return pltpu.async_copy(
                w_hbm.at[idx_v.at[pl.ds(g * CHUNK, CHUNK)]], bufs[b], gsems[b]
            )

        gathers = [fire_gather(g) for g in range(NBUF)]
        for g in range(n_chunks):
            b = g % NBUF
            gathers[b].wait()
            wb = pltpu.async_copy(
                bufs[b], out_hbm.at[pl.ds(base_w + g * CHUNK, CHUNK)], wsem
            )
            wb.wait()
            if g + NBUF < n_chunks:
                gathers[b] = fire_gather(g + NBUF)

    return k


def kernel(idx, W):
    B, L = idx.shape
    V, D = W.shape
    flat = idx.reshape(-1).astype(jnp.int32)
    out = _gather_kernel(B * L, D)(flat, W)
    return out.reshape(B, L, D)
